# baseline (speedup 1.0000x reference)
import dataclasses
import functools

import jax
import jax.numpy as jnp
from jax import lax
from jax.experimental import pallas as pl
from jax.experimental.pallas import tpu as pltpu
from jax.experimental.pallas import tpu_sc as plsc

N_USERS = 5000
N_ITEMS = 5000
N_NODES = N_USERS + N_ITEMS
N_EDGES = 320000
D = 128
N_HOPS = 3

NCORES = 2
NSUB = 16
LANES = 16
DHALF = D // NCORES

ECHUNK = 128
CHUNKS_PER_SUB = 160
NCHUNKS = NSUB * CHUNKS_PER_SUB
E_PAD = NCHUNKS * ECHUNK
NED = 8
NBUF = 4

ROWS_MAIN = 640
ROWS_LAST = N_NODES - 15 * ROWS_MAIN


def _conv_kernel(table_hbm, edata_hbm, zero_hbm, out_hbm, *scr):
    ed = scr[0:NED]
    rows = scr[NED:NED + NBUF]
    sp = scr[NED + NBUF:NED + NBUF + 2]
    base = NED + NBUF + 2
    sem_i = scr[base:base + NBUF]
    sem_g = scr[base + NBUF:base + 2 * NBUF]
    sem_w = scr[base + 2 * NBUF:base + 3 * NBUF]

    c = lax.axis_index("c")
    s = lax.axis_index("s")

    rslice_main = pl.ds(s * ROWS_MAIN, ROWS_MAIN)
    rslice_last = pl.ds(15 * ROWS_MAIN, ROWS_LAST)

    def rowblock_copy(src_ref, dst_ref):
        @pl.when(s < 15)
        def _():
            pltpu.sync_copy(src_ref.at[rslice_main], dst_ref.at[rslice_main])

        @pl.when(s == 15)
        def _():
            pltpu.sync_copy(src_ref.at[rslice_last], dst_ref.at[rslice_last])

    rowblock_copy(table_hbm.at[c], sp[0])
    rowblock_copy(zero_hbm, sp[1])
    plsc.subcore_barrier()

    cbase = s * CHUNKS_PER_SUB

    def fire_meta(k, e):
        pltpu.async_copy(edata_hbm.at[cbase + k], ed[e], sem_i[e % NBUF])

    def wait_meta(k, e):
        pltpu.make_async_copy(edata_hbm.at[cbase + k], ed[e],
                              sem_i[e % NBUF]).wait()

    def scale(e):
        rv, edv = rows[e % NBUF], ed[e]
        two = jnp.full((LANES,), 2, jnp.int32)

        @pl.loop(0, ECHUNK, step=4)
        def _(e0):
            for d in range(4):
                eidx = e0 + d
                ei = jnp.broadcast_to(eidx, (LANES,)).astype(jnp.int32)
                vs = plsc.bitcast(plsc.load_gather(edv, [two, ei]), jnp.float32)
                for j in range(DHALF // LANES):
                    csl = pl.ds(j * LANES, LANES)
                    rv[eidx, csl] = rv[eidx, csl] * vs

    def run_hop(tab, acc):
        def fire_gather(e):
            pltpu.async_copy(tab.at[ed[e].at[0]], rows[e % NBUF],
                             sem_g[e % NBUF])

        def wait_gather(e):
            pltpu.make_async_copy(tab.at[ed[e].at[0]], rows[e % NBUF],
                                  sem_g[e % NBUF]).wait()

        def fire_scatter(e):
            pltpu.async_copy(rows[e % NBUF], acc.at[ed[e].at[1]],
                             sem_w[e % NBUF], add=True)

        def wait_scatter(e):
            pltpu.make_async_copy(rows[e % NBUF], acc.at[ed[e].at[1]],
                                  sem_w[e % NBUF]).wait()

        NITER = CHUNKS_PER_SUB // NED
        for k in range(4):
            fire_meta(k, k)
        wait_meta(0, 0)
        fire_gather(0)
        wait_meta(1, 1)
        fire_gather(1)

        @pl.loop(0, NITER)
        def _(kk):
            k0 = kk * NED
            for u in range(NED):
                k = k0 + u
                eg = (u + 2) % NED
                wait_gather(u)

                if u < 4:
                    fire_meta(k + 4, (u + 4) % NED)
                else:
                    @pl.when(kk <= NITER - 2)
                    def _():
                        fire_meta(k + 4, (u + 4) % NED)

                def advance():
                    wait_meta(k + 2, eg)
                    wait_scatter(eg)
                    fire_gather(eg)

                if u < 2:
                    @pl.when(kk >= 1)
                    def _():
                        advance()

                    @pl.when(kk == 0)
                    def _():
                        wait_meta(k + 2, eg)
                        fire_gather(eg)
                elif u < 6:
                    advance()
                else:
                    @pl.when(kk <= NITER - 2)
                    def _():
                        advance()

                scale(u)
                fire_scatter(u)

        for e in (4, 5, 6, 7):
            wait_scatter(e)

    for hop in range(N_HOPS):
        tab, acc = sp[hop % 2], sp[1 - hop % 2]
        run_hop(tab, acc)
        plsc.subcore_barrier()
        rowblock_copy(acc, out_hbm.at[hop].at[c])
        if hop != N_HOPS - 1:
            rowblock_copy(zero_hbm, tab)
        plsc.subcore_barrier()


@jax.jit
def kernel(user_embed, item_embed, adj_indices, adj_values):
    all_embed = jnp.concatenate([user_embed, item_embed], axis=0)
    pad = E_PAD - N_EDGES
    spread = (jnp.arange(pad, dtype=jnp.int32) * 8) % N_NODES
    dst = jnp.concatenate([adj_indices[0], spread])
    src = jnp.concatenate([adj_indices[1], spread])
    vbits = lax.bitcast_convert_type(
        jnp.concatenate([adj_values, jnp.zeros((pad,), jnp.float32)]),
        jnp.int32)
    edata = jnp.stack([src.reshape(-1, ECHUNK), dst.reshape(-1, ECHUNK),
                       vbits.reshape(-1, ECHUNK)], axis=1)
    zeros = jnp.zeros((N_NODES, DHALF), jnp.float32)

    cp = pltpu.CompilerParams()
    for fld, v in (("needs_layout_passes", False),
                   ("use_tc_tiling_on_sc", False)):
        if fld in pltpu.CompilerParams.__dataclass_fields__:
            cp = dataclasses.replace(cp, **{fld: v})

    mesh = plsc.VectorSubcoreMesh(core_axis_name="c", subcore_axis_name="s")
    conv = pl.kernel(
        _conv_kernel,
        out_type=jax.ShapeDtypeStruct((N_HOPS, NCORES, N_NODES, DHALF),
                                      jnp.float32),
        mesh=mesh,
        compiler_params=cp,
        scratch_types=(
            [pltpu.VMEM((3, ECHUNK), jnp.int32) for _ in range(NED)]
            + [pltpu.VMEM((ECHUNK, DHALF), jnp.float32) for _ in range(NBUF)]
            + [pltpu.VMEM_SHARED((N_NODES, DHALF), jnp.float32)
               for _ in range(2)]
            + [pltpu.SemaphoreType.DMA for _ in range(3 * NBUF)]
        ),
    )

    t = all_embed.reshape(N_NODES, NCORES, DHALF).transpose(1, 0, 2)
    hops = conv(t, edata, zeros)
    embs = [all_embed] + [hops[h].transpose(1, 0, 2).reshape(N_NODES, D)
                          for h in range(N_HOPS)]
    stacked = jnp.stack(embs, axis=1)
    return stacked[:N_USERS], stacked[N_USERS:]


# speedup vs baseline: 1.0003x; 1.0003x over previous
import dataclasses
import functools

import jax
import jax.numpy as jnp
from jax import lax
from jax.experimental import pallas as pl
from jax.experimental.pallas import tpu as pltpu
from jax.experimental.pallas import tpu_sc as plsc

N_USERS = 5000
N_ITEMS = 5000
N_NODES = N_USERS + N_ITEMS
N_EDGES = 320000
D = 128
N_HOPS = 3

NCORES = 2
NSUB = 16
LANES = 16
DHALF = D // NCORES

ECHUNK = 128
CHUNKS_PER_SUB = 160
NCHUNKS = NSUB * CHUNKS_PER_SUB
E_PAD = NCHUNKS * ECHUNK
NED = 16
NBUF = 8

ROWS_MAIN = 640
ROWS_LAST = N_NODES - 15 * ROWS_MAIN


def _conv_kernel(table_hbm, edata_hbm, zero_hbm, out_hbm, *scr):
    ed = scr[0:NED]
    rows = scr[NED:NED + NBUF]
    acc_sh = scr[NED + NBUF]
    base = NED + NBUF + 1
    sem_i = scr[base:base + NBUF]
    sem_g = scr[base + NBUF:base + 2 * NBUF]
    sem_w = scr[base + 2 * NBUF:base + 3 * NBUF]

    c = lax.axis_index("c")
    s = lax.axis_index("s")

    rslice_main = pl.ds(s * ROWS_MAIN, ROWS_MAIN)
    rslice_last = pl.ds(15 * ROWS_MAIN, ROWS_LAST)

    def rowblock_copy(src_ref, dst_ref):
        @pl.when(s < 15)
        def _():
            pltpu.sync_copy(src_ref.at[rslice_main], dst_ref.at[rslice_main])

        @pl.when(s == 15)
        def _():
            pltpu.sync_copy(src_ref.at[rslice_last], dst_ref.at[rslice_last])

    rowblock_copy(zero_hbm, acc_sh)
    plsc.subcore_barrier()

    cbase = s * CHUNKS_PER_SUB

    def fire_meta(k, e):
        pltpu.async_copy(edata_hbm.at[cbase + k], ed[e], sem_i[e % NBUF])

    def wait_meta(k, e):
        pltpu.make_async_copy(edata_hbm.at[cbase + k], ed[e],
                              sem_i[e % NBUF]).wait()

    def scale(e):
        rv, edv = rows[e % NBUF], ed[e]
        two = jnp.full((LANES,), 2, jnp.int32)

        @pl.loop(0, ECHUNK, step=4)
        def _(e0):
            for d in range(4):
                eidx = e0 + d
                ei = jnp.broadcast_to(eidx, (LANES,)).astype(jnp.int32)
                vs = plsc.bitcast(plsc.load_gather(edv, [two, ei]), jnp.float32)
                for j in range(DHALF // LANES):
                    csl = pl.ds(j * LANES, LANES)
                    rv[eidx, csl] = rv[eidx, csl] * vs

    def run_hop(tab, acc):
        def fire_gather(e):
            pltpu.async_copy(tab.at[ed[e].at[0]], rows[e % NBUF],
                             sem_g[e % NBUF])

        def wait_gather(e):
            pltpu.make_async_copy(tab.at[ed[e].at[0]], rows[e % NBUF],
                                  sem_g[e % NBUF]).wait()

        def fire_scatter(e):
            pltpu.async_copy(rows[e % NBUF], acc.at[ed[e].at[1]],
                             sem_w[e % NBUF], add=True)

        def wait_scatter(e):
            pltpu.make_async_copy(rows[e % NBUF], acc.at[ed[e].at[1]],
                                  sem_w[e % NBUF]).wait()

        NITER = CHUNKS_PER_SUB // NED
        for k in range(NBUF):
            fire_meta(k, k)
        for k in range(NBUF // 2):
            wait_meta(k, k)
            fire_gather(k)

        @pl.loop(0, NITER)
        def _(kk):
            k0 = kk * NED
            for u in range(NED):
                k = k0 + u
                eg = (u + 4) % NED
                wait_gather(u)

                if u < NED - NBUF:
                    fire_meta(k + NBUF, (u + NBUF) % NED)
                else:
                    @pl.when(kk <= NITER - 2)
                    def _():
                        fire_meta(k + NBUF, (u + NBUF) % NED)

                def advance():
                    wait_meta(k + 4, eg)
                    wait_scatter(eg)
                    fire_gather(eg)

                if u < 4:
                    @pl.when(kk >= 1)
                    def _():
                        advance()

                    @pl.when(kk == 0)
                    def _():
                        wait_meta(k + 4, eg)
                        fire_gather(eg)
                elif u < NED - 4:
                    advance()
                else:
                    @pl.when(kk <= NITER - 2)
                    def _():
                        advance()

                scale(u)
                fire_scatter(u)

        for e in range(NBUF, 2 * NBUF):
            wait_scatter(e)

    for hop in range(N_HOPS):
        tab = table_hbm.at[c] if hop == 0 else out_hbm.at[hop - 1].at[c]
        run_hop(tab, acc_sh)
        plsc.subcore_barrier()
        rowblock_copy(acc_sh, out_hbm.at[hop].at[c])
        if hop != N_HOPS - 1:
            rowblock_copy(zero_hbm, acc_sh)
        plsc.subcore_barrier()


@jax.jit
def kernel(user_embed, item_embed, adj_indices, adj_values):
    all_embed = jnp.concatenate([user_embed, item_embed], axis=0)
    pad = E_PAD - N_EDGES
    spread = (jnp.arange(pad, dtype=jnp.int32) * 8) % N_NODES
    dst = jnp.concatenate([adj_indices[0], spread])
    src = jnp.concatenate([adj_indices[1], spread])
    vbits = lax.bitcast_convert_type(
        jnp.concatenate([adj_values, jnp.zeros((pad,), jnp.float32)]),
        jnp.int32)
    edata = jnp.stack([src.reshape(-1, ECHUNK), dst.reshape(-1, ECHUNK),
                       vbits.reshape(-1, ECHUNK)], axis=1)
    zeros = jnp.zeros((N_NODES, DHALF), jnp.float32)

    cp = pltpu.CompilerParams()
    for fld, v in (("needs_layout_passes", False),
                   ("use_tc_tiling_on_sc", False)):
        if fld in pltpu.CompilerParams.__dataclass_fields__:
            cp = dataclasses.replace(cp, **{fld: v})

    mesh = plsc.VectorSubcoreMesh(core_axis_name="c", subcore_axis_name="s")
    conv = pl.kernel(
        _conv_kernel,
        out_type=jax.ShapeDtypeStruct((N_HOPS, NCORES, N_NODES, DHALF),
                                      jnp.float32),
        mesh=mesh,
        compiler_params=cp,
        scratch_types=(
            [pltpu.VMEM((3, ECHUNK), jnp.int32) for _ in range(NED)]
            + [pltpu.VMEM((ECHUNK, DHALF), jnp.float32) for _ in range(NBUF)]
            + [pltpu.VMEM_SHARED((N_NODES, DHALF), jnp.float32)]
            + [pltpu.SemaphoreType.DMA for _ in range(3 * NBUF)]
        ),
    )

    t = all_embed.reshape(N_NODES, NCORES, DHALF).transpose(1, 0, 2)
    hops = conv(t, edata, zeros)
    embs = [all_embed] + [hops[h].transpose(1, 0, 2).reshape(N_NODES, D)
                          for h in range(N_HOPS)]
    stacked = jnp.stack(embs, axis=1)
    return stacked[:N_USERS], stacked[N_USERS:]


# speedup vs baseline: 1.0620x; 1.0617x over previous
import dataclasses
import functools

import jax
import jax.numpy as jnp
from jax import lax
from jax.experimental import pallas as pl
from jax.experimental.pallas import tpu as pltpu
from jax.experimental.pallas import tpu_sc as plsc

N_USERS = 5000
N_ITEMS = 5000
N_NODES = N_USERS + N_ITEMS
N_EDGES = 320000
D = 128
N_HOPS = 3

NCORES = 2
NSUB = 16
LANES = 16

ECHUNK = 128
CHUNKS_PER_SUB = 80
NCHUNKS = NCORES * NSUB * CHUNKS_PER_SUB
E_PAD = NCHUNKS * ECHUNK
NED = 4
NBUF = 2

ROWS_MAIN = 640
ROWS_LAST = N_NODES - 15 * ROWS_MAIN


def _hop_kernel(table_hbm, edata_hbm, zero_hbm, out_hbm, *scr):
    ed = scr[0:NED]
    rows = scr[NED:NED + NBUF]
    acc_sh = scr[NED + NBUF]
    base = NED + NBUF + 1
    sem_i = scr[base:base + NBUF]
    sem_g = scr[base + NBUF:base + 2 * NBUF]
    sem_w = scr[base + 2 * NBUF:base + 3 * NBUF]

    c = lax.axis_index("c")
    s = lax.axis_index("s")

    rslice_main = pl.ds(s * ROWS_MAIN, ROWS_MAIN)
    rslice_last = pl.ds(15 * ROWS_MAIN, ROWS_LAST)

    def rowblock_copy(src_ref, dst_ref):
        @pl.when(s < 15)
        def _():
            pltpu.sync_copy(src_ref.at[rslice_main], dst_ref.at[rslice_main])

        @pl.when(s == 15)
        def _():
            pltpu.sync_copy(src_ref.at[rslice_last], dst_ref.at[rslice_last])

    rowblock_copy(zero_hbm, acc_sh)
    plsc.subcore_barrier()

    cbase = (c * NSUB + s) * CHUNKS_PER_SUB

    def fire_meta(k, e):
        pltpu.async_copy(edata_hbm.at[cbase + k], ed[e], sem_i[e % NBUF])

    def wait_meta(k, e):
        pltpu.make_async_copy(edata_hbm.at[cbase + k], ed[e],
                              sem_i[e % NBUF]).wait()

    def fire_gather(e):
        pltpu.async_copy(table_hbm.at[ed[e].at[0]], rows[e % NBUF],
                         sem_g[e % NBUF])

    def wait_gather(e):
        pltpu.make_async_copy(table_hbm.at[ed[e].at[0]], rows[e % NBUF],
                              sem_g[e % NBUF]).wait()

    def fire_scatter(e):
        pltpu.async_copy(rows[e % NBUF], acc_sh.at[ed[e].at[1]],
                         sem_w[e % NBUF], add=True)

    def wait_scatter(e):
        pltpu.make_async_copy(rows[e % NBUF], acc_sh.at[ed[e].at[1]],
                              sem_w[e % NBUF]).wait()

    def scale(e):
        rv, edv = rows[e % NBUF], ed[e]
        two = jnp.full((LANES,), 2, jnp.int32)

        @pl.loop(0, ECHUNK, step=2)
        def _(e0):
            for d in range(2):
                eidx = e0 + d
                ei = jnp.broadcast_to(eidx, (LANES,)).astype(jnp.int32)
                vs = plsc.bitcast(plsc.load_gather(edv, [two, ei]), jnp.float32)
                for j in range(D // LANES):
                    csl = pl.ds(j * LANES, LANES)
                    rv[eidx, csl] = rv[eidx, csl] * vs

    NITER = CHUNKS_PER_SUB // NED
    fire_meta(0, 0)
    fire_meta(1, 1)
    wait_meta(0, 0)
    fire_gather(0)

    @pl.loop(0, NITER)
    def _(kk):
        k0 = kk * NED
        for u in range(NED):
            k = k0 + u
            en = (u + 1) % NED
            wait_gather(u)

            if u < 2:
                fire_meta(k + 2, (u + 2) % NED)
            else:
                @pl.when(kk <= NITER - 2)
                def _():
                    fire_meta(k + 2, (u + 2) % NED)

            def advance():
                wait_meta(k + 1, en)
                wait_scatter(en)
                fire_gather(en)

            if u == 0:
                @pl.when(kk >= 1)
                def _():
                    advance()

                @pl.when(kk == 0)
                def _():
                    wait_meta(k + 1, en)
                    fire_gather(en)
            elif u < 3:
                advance()
            else:
                @pl.when(kk <= NITER - 2)
                def _():
                    advance()

            scale(u)
            fire_scatter(u)

    wait_scatter(2)
    wait_scatter(3)

    plsc.subcore_barrier()
    rowblock_copy(acc_sh, out_hbm.at[c])


@jax.jit
def kernel(user_embed, item_embed, adj_indices, adj_values):
    all_embed = jnp.concatenate([user_embed, item_embed], axis=0)
    pad = E_PAD - N_EDGES
    spread = (jnp.arange(pad, dtype=jnp.int32) * 8) % N_NODES
    dst = jnp.concatenate([adj_indices[0], spread])
    src = jnp.concatenate([adj_indices[1], spread])
    vbits = lax.bitcast_convert_type(
        jnp.concatenate([adj_values, jnp.zeros((pad,), jnp.float32)]),
        jnp.int32)
    edata = jnp.stack([src.reshape(-1, ECHUNK), dst.reshape(-1, ECHUNK),
                       vbits.reshape(-1, ECHUNK)], axis=1)
    zeros = jnp.zeros((N_NODES, D), jnp.float32)

    cp = pltpu.CompilerParams()
    for fld, v in (("needs_layout_passes", False),
                   ("use_tc_tiling_on_sc", False)):
        if fld in pltpu.CompilerParams.__dataclass_fields__:
            cp = dataclasses.replace(cp, **{fld: v})

    mesh = plsc.VectorSubcoreMesh(core_axis_name="c", subcore_axis_name="s")
    hop = pl.kernel(
        _hop_kernel,
        out_type=jax.ShapeDtypeStruct((NCORES, N_NODES, D), jnp.float32),
        mesh=mesh,
        compiler_params=cp,
        scratch_types=(
            [pltpu.VMEM((3, ECHUNK), jnp.int32) for _ in range(NED)]
            + [pltpu.VMEM((ECHUNK, D), jnp.float32) for _ in range(NBUF)]
            + [pltpu.VMEM_SHARED((N_NODES, D), jnp.float32)]
            + [pltpu.SemaphoreType.DMA for _ in range(3 * NBUF)]
        ),
    )

    t = all_embed
    embs = [all_embed]
    for _ in range(N_HOPS):
        parts = hop(t, edata, zeros)
        t = parts[0] + parts[1]
        embs.append(t)
    stacked = jnp.stack(embs, axis=1)
    return stacked[:N_USERS], stacked[N_USERS:]


# speedup vs baseline: 1.2204x; 1.1491x over previous
import dataclasses
import functools

import jax
import jax.numpy as jnp
from jax import lax
from jax.experimental import pallas as pl
from jax.experimental.pallas import tpu as pltpu
from jax.experimental.pallas import tpu_sc as plsc

N_USERS = 5000
N_ITEMS = 5000
N_NODES = N_USERS + N_ITEMS
N_EDGES = 320000
D = 128
N_HOPS = 3

NCORES = 2
NSUB = 16
LANES = 16

ECHUNK = 128
CHUNKS_PER_SUB = 80
NCHUNKS = NCORES * NSUB * CHUNKS_PER_SUB
E_PAD = NCHUNKS * ECHUNK
NED = 4
NBUF = 2

ROWS_MAIN = 640
ROWS_LAST = N_NODES - 15 * ROWS_MAIN


def _hop_kernel(table_hbm, edata_hbm, zero_hbm, out_hbm, *scr):
    ed = scr[0:NED]
    rows = scr[NED:NED + NBUF]
    acc_sh = scr[NED + NBUF]
    base = NED + NBUF + 1
    sem_i = scr[base:base + NBUF]
    sem_g = scr[base + NBUF:base + 2 * NBUF]
    sem_w = scr[base + 2 * NBUF:base + 3 * NBUF]

    c = lax.axis_index("c")
    s = lax.axis_index("s")

    rslice_main = pl.ds(s * ROWS_MAIN, ROWS_MAIN)
    rslice_last = pl.ds(15 * ROWS_MAIN, ROWS_LAST)

    def rowblock_copy(src_ref, dst_ref):
        @pl.when(s < 15)
        def _():
            pltpu.sync_copy(src_ref.at[rslice_main], dst_ref.at[rslice_main])

        @pl.when(s == 15)
        def _():
            pltpu.sync_copy(src_ref.at[rslice_last], dst_ref.at[rslice_last])

    rowblock_copy(zero_hbm, acc_sh)
    plsc.subcore_barrier()

    cbase = (c * NSUB + s) * CHUNKS_PER_SUB

    def fire_meta(k, e):
        pltpu.async_copy(edata_hbm.at[cbase + k], ed[e], sem_i[e % NBUF])

    def wait_meta(k, e):
        pltpu.make_async_copy(edata_hbm.at[cbase + k], ed[e],
                              sem_i[e % NBUF]).wait()

    def fire_gather(e):
        pltpu.async_copy(table_hbm.at[ed[e].at[0]], rows[e % NBUF],
                         sem_g[e % NBUF])

    def wait_gather(e):
        pltpu.make_async_copy(table_hbm.at[ed[e].at[0]], rows[e % NBUF],
                              sem_g[e % NBUF]).wait()

    def fire_scatter(e):
        pltpu.async_copy(rows[e % NBUF], acc_sh.at[ed[e].at[1]],
                         sem_w[e % NBUF], add=True)

    def wait_scatter(e):
        pltpu.make_async_copy(rows[e % NBUF], acc_sh.at[ed[e].at[1]],
                              sem_w[e % NBUF]).wait()

    gdn = lax.GatherDimensionNumbers(offset_dims=(),
                                     collapsed_slice_dims=(0,),
                                     start_index_map=(0,))

    def scale(e):
        rv, edv = rows[e % NBUF], ed[e]

        @pl.loop(0, ECHUNK, step=LANES)
        def _(e0):
            vv = plsc.bitcast(edv[2, pl.ds(e0, LANES)], jnp.float32)
            for d in range(LANES):
                eidx = e0 + d
                vs = lax.gather(vv, jnp.full((LANES, 1), d, jnp.int32), gdn,
                                (1,),
                                mode=lax.GatherScatterMode.PROMISE_IN_BOUNDS)
                for j in range(D // LANES):
                    csl = pl.ds(j * LANES, LANES)
                    rv[eidx, csl] = rv[eidx, csl] * vs

    NITER = CHUNKS_PER_SUB // NED
    fire_meta(0, 0)
    fire_meta(1, 1)
    wait_meta(0, 0)
    fire_gather(0)

    @pl.loop(0, NITER)
    def _(kk):
        k0 = kk * NED
        for u in range(NED):
            k = k0 + u
            en = (u + 1) % NED
            wait_gather(u)

            if u < 2:
                fire_meta(k + 2, (u + 2) % NED)
            else:
                @pl.when(kk <= NITER - 2)
                def _():
                    fire_meta(k + 2, (u + 2) % NED)

            def advance():
                wait_meta(k + 1, en)
                wait_scatter(en)
                fire_gather(en)

            if u == 0:
                @pl.when(kk >= 1)
                def _():
                    advance()

                @pl.when(kk == 0)
                def _():
                    wait_meta(k + 1, en)
                    fire_gather(en)
            elif u < 3:
                advance()
            else:
                @pl.when(kk <= NITER - 2)
                def _():
                    advance()

            scale(u)
            fire_scatter(u)

    wait_scatter(2)
    wait_scatter(3)

    plsc.subcore_barrier()
    rowblock_copy(acc_sh, out_hbm.at[c])


@jax.jit
def kernel(user_embed, item_embed, adj_indices, adj_values):
    all_embed = jnp.concatenate([user_embed, item_embed], axis=0)
    pad = E_PAD - N_EDGES
    spread = (jnp.arange(pad, dtype=jnp.int32) * 8) % N_NODES
    dst = jnp.concatenate([adj_indices[0], spread])
    src = jnp.concatenate([adj_indices[1], spread])
    vbits = lax.bitcast_convert_type(
        jnp.concatenate([adj_values, jnp.zeros((pad,), jnp.float32)]),
        jnp.int32)
    edata = jnp.stack([src.reshape(-1, ECHUNK), dst.reshape(-1, ECHUNK),
                       vbits.reshape(-1, ECHUNK)], axis=1)
    zeros = jnp.zeros((N_NODES, D), jnp.float32)

    cp = pltpu.CompilerParams()
    for fld, v in (("needs_layout_passes", False),
                   ("use_tc_tiling_on_sc", False)):
        if fld in pltpu.CompilerParams.__dataclass_fields__:
            cp = dataclasses.replace(cp, **{fld: v})

    mesh = plsc.VectorSubcoreMesh(core_axis_name="c", subcore_axis_name="s")
    hop = pl.kernel(
        _hop_kernel,
        out_type=jax.ShapeDtypeStruct((NCORES, N_NODES, D), jnp.float32),
        mesh=mesh,
        compiler_params=cp,
        scratch_types=(
            [pltpu.VMEM((3, ECHUNK), jnp.int32) for _ in range(NED)]
            + [pltpu.VMEM((ECHUNK, D), jnp.float32) for _ in range(NBUF)]
            + [pltpu.VMEM_SHARED((N_NODES, D), jnp.float32)]
            + [pltpu.SemaphoreType.DMA for _ in range(3 * NBUF)]
        ),
    )

    t = all_embed
    embs = [all_embed]
    for _ in range(N_HOPS):
        parts = hop(t, edata, zeros)
        t = parts[0] + parts[1]
        embs.append(t)
    stacked = jnp.stack(embs, axis=1)
    return stacked[:N_USERS], stacked[N_USERS:]


# speedup vs baseline: 1.2319x; 1.0095x over previous
import dataclasses

import jax
import jax.numpy as jnp
from jax import lax
from jax.experimental import pallas as pl
from jax.experimental.pallas import tpu as pltpu
from jax.experimental.pallas import tpu_sc as plsc

N_USERS = 5000
N_ITEMS = 5000
N_NODES = N_USERS + N_ITEMS
N_EDGES = 320000
D = 128
N_HOPS = 3

NCORES = 2
NSUB = 16
LANES = 16

ECHUNK = 128
CHUNKS_PER_SUB = 80
NCHUNKS = NCORES * NSUB * CHUNKS_PER_SUB
E_PAD = NCHUNKS * ECHUNK
NED = 4
NBUF = 2

ROWS_MAIN = 640
ROWS_LAST = N_NODES - 15 * ROWS_MAIN


def _hop_kernel(table_hbm, edata_hbm, zero_hbm, out_hbm, *scr):
    ed = scr[0:NED]
    rows = scr[NED:NED + NBUF]
    acc_sh = scr[NED + NBUF]
    base = NED + NBUF + 1
    sem_i = scr[base:base + NBUF]
    sem_g = scr[base + NBUF:base + 2 * NBUF]
    sem_w = scr[base + 2 * NBUF:base + 3 * NBUF]

    c = lax.axis_index("c")
    s = lax.axis_index("s")

    rslice_main = pl.ds(s * ROWS_MAIN, ROWS_MAIN)
    rslice_last = pl.ds(15 * ROWS_MAIN, ROWS_LAST)

    def rowblock_copy(src_ref, dst_ref):
        @pl.when(s < 15)
        def _():
            pltpu.sync_copy(src_ref.at[rslice_main], dst_ref.at[rslice_main])

        @pl.when(s == 15)
        def _():
            pltpu.sync_copy(src_ref.at[rslice_last], dst_ref.at[rslice_last])

    rowblock_copy(zero_hbm, acc_sh)
    plsc.subcore_barrier()

    cbase = (c * NSUB + s) * CHUNKS_PER_SUB

    def fire_meta(k, e):
        pltpu.async_copy(edata_hbm.at[cbase + k], ed[e], sem_i[e % NBUF])

    def wait_meta(k, e):
        pltpu.make_async_copy(edata_hbm.at[cbase + k], ed[e],
                              sem_i[e % NBUF]).wait()

    def fire_gather(e):
        pltpu.async_copy(table_hbm.at[ed[e].at[0]], rows[e % NBUF],
                         sem_g[e % NBUF])

    def wait_gather(e):
        pltpu.make_async_copy(table_hbm.at[ed[e].at[0]], rows[e % NBUF],
                              sem_g[e % NBUF]).wait()

    def fire_scatter(e):
        pltpu.async_copy(rows[e % NBUF], acc_sh.at[ed[e].at[1]],
                         sem_w[e % NBUF], add=True)

    def wait_scatter(e):
        pltpu.make_async_copy(rows[e % NBUF], acc_sh.at[ed[e].at[1]],
                              sem_w[e % NBUF]).wait()

    gdn = lax.GatherDimensionNumbers(offset_dims=(),
                                     collapsed_slice_dims=(0,),
                                     start_index_map=(0,))

    def scale(e):
        rv, edv = rows[e % NBUF], ed[e]

        @pl.loop(0, ECHUNK, step=LANES)
        def _(e0):
            vv = plsc.bitcast(edv[2, pl.ds(e0, LANES)], jnp.float32)
            for d in range(LANES):
                eidx = e0 + d
                vs = lax.gather(vv, jnp.full((LANES, 1), d, jnp.int32), gdn,
                                (1,),
                                mode=lax.GatherScatterMode.PROMISE_IN_BOUNDS)
                for j in range(D // LANES):
                    csl = pl.ds(j * LANES, LANES)
                    rv[eidx, csl] = rv[eidx, csl] * vs

    NITER = CHUNKS_PER_SUB // NED
    fire_meta(0, 0)
    fire_meta(1, 1)
    wait_meta(0, 0)
    fire_gather(0)

    @pl.loop(0, NITER)
    def _(kk):
        k0 = kk * NED
        for u in range(NED):
            k = k0 + u
            en = (u + 1) % NED
            wait_gather(u)

            if u < 2:
                fire_meta(k + 2, (u + 2) % NED)
            else:
                @pl.when(kk <= NITER - 2)
                def _():
                    fire_meta(k + 2, (u + 2) % NED)

            def advance():
                wait_meta(k + 1, en)
                wait_scatter(en)
                fire_gather(en)

            if u == 0:
                @pl.when(kk >= 1)
                def _():
                    advance()

                @pl.when(kk == 0)
                def _():
                    wait_meta(k + 1, en)
                    fire_gather(en)
            elif u < 3:
                advance()
            else:
                @pl.when(kk <= NITER - 2)
                def _():
                    advance()

            scale(u)
            fire_scatter(u)

    wait_scatter(2)
    wait_scatter(3)

    plsc.subcore_barrier()
    rowblock_copy(acc_sh, out_hbm.at[c])


@jax.jit
def kernel(user_embed, item_embed, adj_indices, adj_values):
    all_embed = jnp.concatenate([user_embed, item_embed], axis=0)
    pad = E_PAD - N_EDGES
    spread = (jnp.arange(pad, dtype=jnp.int32) * 8) % N_NODES
    dst = jnp.concatenate([adj_indices[0], spread])
    src = jnp.concatenate([adj_indices[1], spread])
    vbits = lax.bitcast_convert_type(
        jnp.concatenate([adj_values, jnp.zeros((pad,), jnp.float32)]),
        jnp.int32)
    edata = jnp.stack([src.reshape(-1, ECHUNK), dst.reshape(-1, ECHUNK),
                       vbits.reshape(-1, ECHUNK)], axis=1)
    zeros = jnp.zeros((N_NODES, D), jnp.float32)

    cp = pltpu.CompilerParams()
    for fld, v in (("needs_layout_passes", False),
                   ("use_tc_tiling_on_sc", False)):
        if fld in pltpu.CompilerParams.__dataclass_fields__:
            cp = dataclasses.replace(cp, **{fld: v})

    mesh = plsc.VectorSubcoreMesh(core_axis_name="c", subcore_axis_name="s")
    hop = pl.kernel(
        _hop_kernel,
        out_type=jax.ShapeDtypeStruct((NCORES, N_NODES, D), jnp.float32),
        mesh=mesh,
        compiler_params=cp,
        scratch_types=(
            [pltpu.VMEM((3, ECHUNK), jnp.int32) for _ in range(NED)]
            + [pltpu.VMEM((ECHUNK, D), jnp.float32) for _ in range(NBUF)]
            + [pltpu.VMEM_SHARED((N_NODES, D), jnp.float32)]
            + [pltpu.SemaphoreType.DMA for _ in range(3 * NBUF)]
        ),
    )

    t = all_embed
    embs = [all_embed]
    for _ in range(N_HOPS):
        parts = hop(t, edata, zeros)
        t = parts[0] + parts[1]
        embs.append(t)
    stacked = jnp.stack(embs, axis=1)
    return stacked[:N_USERS], stacked[N_USERS:]
